# baseline (speedup 1.0000x reference)
import jax
import jax.numpy as jnp
from jax import lax
from jax.experimental import pallas as pl
from jax.experimental.pallas import tpu as pltpu
from jax.experimental.pallas import tpu_sc as plsc

N_CAT = 26
VOCAB = 100000
EMB = 64
N_NUM = 13
BATCH = 16384

NC = 2
NS = 16
NW = NC * NS
E_PER_W = EMB // NW
OUT_CHUNK = 4096
L = 16


VOCAB_AL = 99968
VCB = (0, 40960, 81920)
VCS_DMA = (40960, 40960, VOCAB_AL - 81920)
VCS = (40960, 40960, VOCAB - 81920)
VBUF = 40960
TAILP = 128
UNITS = [(de, c) for de in range(E_PER_W) for c in range(len(VCB))]


def _sc_gather_body(xcat_t, tables_t, tail_t, cat_t, idx_v, rv0, rv1, out_v,
                    sem0, sem1):
    wid = lax.axis_index("s") * NC + lax.axis_index("c")
    sems = (sem0, sem1)
    rvs = (rv0, rv1)

    @pl.loop(0, N_CAT)
    def _field(i):
        pltpu.sync_copy(xcat_t.at[i], idx_v)

        def start(u):
            de, c = UNITS[u]
            b = u % 2
            e = wid * E_PER_W + de
            cp = [
                pltpu.async_copy(
                    tables_t.at[i, e, pl.ds(VCB[c], VCS_DMA[c])],
                    rvs[b].at[pl.ds(0, VCS_DMA[c])],
                    sems[b],
                )
            ]
            if c == len(VCB) - 1:
                cp.append(
                    pltpu.async_copy(
                        tail_t.at[i, e],
                        rvs[b].at[pl.ds(VCS_DMA[c], TAILP)],
                        sems[b],
                    )
                )
            return cp

        cps = [None] * len(UNITS)
        cps[0] = start(0)
        for u, (de, c) in enumerate(UNITS):
            if u + 1 < len(UNITS):
                cps[u + 1] = start(u + 1)
            for cp in cps[u]:
                cp.wait()
            b = u % 2
            base, size = VCB[c], VCS[c]

            @pl.loop(0, BATCH // L, unroll=8)
            def _vec(k):
                idxv = idx_v[pl.ds(k * L, L)]
                t = idxv - base
                tc = jnp.minimum(jnp.maximum(t, 0), size - 1)
                g = plsc.load_gather(rvs[b], [tc])
                val = jnp.where((t >= 0) & (t < size), g, 0.0)
                if c == 0:
                    out_v[pl.ds(k * L, L)] = val
                else:
                    plsc.addupdate(out_v.at[pl.ds(k * L, L)], val)

            if c == len(VCB) - 1:
                pltpu.sync_copy(out_v, cat_t.at[i, wid * E_PER_W + de])


_sc_gather = pl.kernel(
    _sc_gather_body,
    out_type=jax.ShapeDtypeStruct((N_CAT, EMB, BATCH), jnp.float32),
    mesh=plsc.VectorSubcoreMesh(
        core_axis_name="c", subcore_axis_name="s", num_cores=NC, num_subcores=NS
    ),
    scratch_types=[
        pltpu.VMEM((BATCH,), jnp.int32),
        pltpu.VMEM((VBUF,), jnp.float32),
        pltpu.VMEM((VBUF,), jnp.float32),
        pltpu.VMEM((BATCH,), jnp.float32),
        pltpu.SemaphoreType.DMA,
        pltpu.SemaphoreType.DMA,
    ],
    compiler_params=pltpu.CompilerParams(
        use_tc_tiling_on_sc=True, needs_layout_passes=False
    ),
)

BB = 2048


def _tc_proj_body(cat_ref, xn_ref, wn_ref, bn_ref, wfT_ref, bf_ref, out_ref):
    num = jnp.dot(xn_ref[...], wn_ref[...], preferred_element_type=jnp.float32)
    num = num + jnp.sum(bn_ref[...], axis=0, keepdims=True)
    acc = jnp.dot(num, wfT_ref[N_CAT * EMB :, :], preferred_element_type=jnp.float32)
    for i in range(N_CAT):
        acc = acc + lax.dot_general(
            cat_ref[i], wfT_ref[i * EMB : (i + 1) * EMB, :],
            dimension_numbers=(((0,), (0,)), ((), ())),
            preferred_element_type=jnp.float32,
        )
    out_ref[...] = acc + bf_ref[...]


_tc_proj = pl.pallas_call(
    _tc_proj_body,
    grid=(BATCH // BB,),
    in_specs=[
        pl.BlockSpec((N_CAT, EMB, BB), lambda b: (0, 0, b)),
        pl.BlockSpec((BB, N_NUM), lambda b: (b, 0)),
        pl.BlockSpec((N_NUM, EMB), lambda b: (0, 0)),
        pl.BlockSpec((N_NUM, EMB), lambda b: (0, 0)),
        pl.BlockSpec((N_CAT * EMB + EMB, EMB), lambda b: (0, 0)),
        pl.BlockSpec((1, EMB), lambda b: (0, 0)),
    ],
    out_specs=pl.BlockSpec((BB, EMB), lambda b: (b, 0)),
    out_shape=jax.ShapeDtypeStruct((BATCH, EMB), jnp.float32),
)


def kernel(X_cat, X_num, tables, W_num, b_num, W_final, b_final):
    tables_t = tables.transpose(0, 2, 1)
    xcat_t = X_cat.T
    tail_t = jnp.pad(
        tables[:, VOCAB_AL:, :].transpose(0, 2, 1),
        ((0, 0), (0, 0), (0, TAILP - (VOCAB - VOCAB_AL))),
    )
    cat_t = _sc_gather(xcat_t, tables_t, tail_t)
    return _tc_proj(
        cat_t, X_num, W_num, b_num, W_final.T, b_final.reshape(1, EMB)
    )


# speedup vs baseline: 1.6804x; 1.6804x over previous
import jax
import jax.numpy as jnp
from jax import lax
from jax.experimental import pallas as pl
from jax.experimental.pallas import tpu as pltpu
from jax.experimental.pallas import tpu_sc as plsc

N_CAT = 26
VOCAB = 100000
EMB = 64
N_NUM = 13
BATCH = 16384

NC = 2
NS = 16
NW = NC * NS
E_PER_W = EMB // NW
OUT_CHUNK = 4096
L = 16


NCHUNK = BATCH // OUT_CHUNK


def _sc_gather_body(xcat_t, tables_t, cat_t, idx_v, row_v, out_v, sem_row, sem_out):
    wid = lax.axis_index("s") * NC + lax.axis_index("c")

    @pl.loop(0, N_CAT)
    def _field(i):
        pltpu.sync_copy(xcat_t.at[i], idx_v)
        for de in range(E_PER_W):
            e = wid * E_PER_W + de
            pltpu.async_copy(tables_t.at[i, e], row_v, sem_row).wait()

            pending = [None, None]
            for c in range(NCHUNK):
                b = c % 2
                if pending[b] is not None:
                    pending[b].wait()
                    pending[b] = None

                @pl.loop(0, OUT_CHUNK // L, unroll=8)
                def _vec(k):
                    idxv = idx_v[pl.ds(c * OUT_CHUNK + k * L, L)]
                    out_v[b, pl.ds(k * L, L)] = plsc.load_gather(row_v, [idxv])

                pending[b] = pltpu.async_copy(
                    out_v.at[b],
                    cat_t.at[i, e, pl.ds(c * OUT_CHUNK, OUT_CHUNK)],
                    sem_out,
                )
            for b in range(2):
                if pending[b] is not None:
                    pending[b].wait()


_sc_gather = pl.kernel(
    _sc_gather_body,
    out_type=jax.ShapeDtypeStruct((N_CAT, EMB, BATCH), jnp.float32),
    mesh=plsc.VectorSubcoreMesh(
        core_axis_name="c", subcore_axis_name="s", num_cores=NC, num_subcores=NS
    ),
    scratch_types=[
        pltpu.VMEM((BATCH,), jnp.int32),
        pltpu.VMEM((VOCAB,), jnp.float32),
        pltpu.VMEM((2, OUT_CHUNK), jnp.float32),
        pltpu.SemaphoreType.DMA,
        pltpu.SemaphoreType.DMA,
    ],
    compiler_params=pltpu.CompilerParams(
        use_tc_tiling_on_sc=True, needs_layout_passes=False
    ),
)

BB = 2048


def _tc_proj_body(cat_ref, xn_ref, wn_ref, bn_ref, wfT_ref, bf_ref, out_ref):
    num = jnp.dot(xn_ref[...], wn_ref[...], preferred_element_type=jnp.float32)
    num = num + jnp.sum(bn_ref[...], axis=0, keepdims=True)
    acc = jnp.dot(num, wfT_ref[N_CAT * EMB :, :], preferred_element_type=jnp.float32)
    for i in range(N_CAT):
        acc = acc + lax.dot_general(
            cat_ref[i], wfT_ref[i * EMB : (i + 1) * EMB, :],
            dimension_numbers=(((0,), (0,)), ((), ())),
            preferred_element_type=jnp.float32,
        )
    out_ref[...] = acc + bf_ref[...]


_tc_proj = pl.pallas_call(
    _tc_proj_body,
    grid=(BATCH // BB,),
    in_specs=[
        pl.BlockSpec((N_CAT, EMB, BB), lambda b: (0, 0, b)),
        pl.BlockSpec((BB, N_NUM), lambda b: (b, 0)),
        pl.BlockSpec((N_NUM, EMB), lambda b: (0, 0)),
        pl.BlockSpec((N_NUM, EMB), lambda b: (0, 0)),
        pl.BlockSpec((N_CAT * EMB + EMB, EMB), lambda b: (0, 0)),
        pl.BlockSpec((1, EMB), lambda b: (0, 0)),
    ],
    out_specs=pl.BlockSpec((BB, EMB), lambda b: (b, 0)),
    out_shape=jax.ShapeDtypeStruct((BATCH, EMB), jnp.float32),
)


def kernel(X_cat, X_num, tables, W_num, b_num, W_final, b_final):
    tables_t = tables.transpose(0, 2, 1)
    xcat_t = X_cat.T
    cat_t = _sc_gather(xcat_t, tables_t)
    return _tc_proj(
        cat_t, X_num, W_num, b_num, W_final.T, b_final.reshape(1, EMB)
    )
